# baseline (speedup 1.0000x reference)
import functools

import jax
import jax.numpy as jnp
from jax import lax
from jax.experimental import pallas as pl
from jax.experimental.pallas import tpu as pltpu
from jax.experimental.pallas import tpu_sc as plsc

_TPD = 24.0


def _ctx_add_body(ctx_ref, pe_ref, out_ref):
    out_ref[...] = ctx_ref[...] + pe_ref[...]


def _ctx_add(context_tokens, positional_encoding, bn):
    n, b, d = context_tokens.shape
    return pl.pallas_call(
        _ctx_add_body,
        grid=(n // bn,),
        in_specs=[
            pl.BlockSpec((bn, b, d), lambda i: (i, 0, 0)),
            pl.BlockSpec((bn, 1, d), lambda i: (i, 0, 0)),
        ],
        out_specs=pl.BlockSpec((bn, b, d), lambda i: (i, 0, 0)),
        out_shape=jax.ShapeDtypeStruct((n, b, d), jnp.float32),
    )(context_tokens, positional_encoding)


def _make_tgt_kernel(B, D):
    L = 16
    n_workers = B // L
    mesh = plsc.VectorSubcoreMesh(
        core_axis_name="c", subcore_axis_name="s", num_cores=1)

    @functools.partial(
        pl.kernel,
        mesh=mesh,
        out_type=jax.ShapeDtypeStruct((B, D), jnp.float32),
        scratch_types=[
            pltpu.VMEM((4 * L,), jnp.float32),
            pltpu.VMEM((L,), jnp.int32),
            pltpu.VMEM((L, D), jnp.float32),
            pltpu.VMEM((L, D), jnp.float32),
            pltpu.SemaphoreType.DMA,
            pltpu.SemaphoreType.DMA,
        ],
    )
    def tgt_kernel(bbox_hbm, tgt_hbm, pe_hbm, out_hbm,
                   bbox_v, idx_v, rows_v, tgt_v, sem_g, sem_t):
        wid = lax.axis_index("s")

        @pl.when(wid < n_workers)
        def _():
            base = wid * L
            tcp = pltpu.async_copy(tgt_hbm.at[pl.ds(base, L)], tgt_v, sem_t)
            pltpu.async_copy(
                bbox_hbm.at[pl.ds(base * 4, 4 * L)], bbox_v, sem_g).wait()
            vs = [bbox_v[pl.ds(j * L, L)] for j in range(4)]
            lanes = lax.iota(jnp.int32, L)
            four = jnp.full((L,), 4, jnp.int32)
            sel = lax.div(lanes, four)
            lmod = lax.rem(lanes, four)

            def comp(c):
                src = lmod * 4 + c
                g = [v.at[src].get(mode="promise_in_bounds") for v in vs]
                return jnp.where(
                    sel == 0, g[0],
                    jnp.where(sel == 1, g[1],
                              jnp.where(sel == 2, g[2], g[3])))

            x = comp(0)
            y = comp(1)
            w = comp(2)
            h = comp(3)
            cx = (x + w / 2.0) * _TPD
            cy = (y + h / 2.0) * _TPD
            tx = cx.astype(jnp.int32)
            ty = cy.astype(jnp.int32)
            ix = jnp.where(cx > tx.astype(jnp.float32), tx, tx - 1)
            iy = jnp.where(cy > ty.astype(jnp.float32), ty, ty - 1)
            idx_v[...] = ix + iy * 24

            pltpu.async_copy(pe_hbm.at[idx_v], rows_v, sem_g).wait()
            tcp.wait()

            n_chunks = L * D // L
            UNROLL = 1

            def add_body(i, _):
                for u in range(UNROLL):
                    k = i * UNROLL + u
                    r = k // (D // L)
                    off = (k % (D // L)) * L
                    rows_v[r, pl.ds(off, L)] = (
                        rows_v[r, pl.ds(off, L)] + tgt_v[r, pl.ds(off, L)]
                    )
                return 0
            lax.fori_loop(0, n_chunks // UNROLL, add_body, 0)

            pltpu.sync_copy(rows_v, out_hbm.at[pl.ds(base, L)])

    return tgt_kernel


def kernel(context_tokens, target_tokens, target_bbox, positional_encoding):
    n, b, d = context_tokens.shape
    pe2d = positional_encoding.reshape(n, d)
    tgt2d = target_tokens.reshape(b, d)
    tgt_out = _make_tgt_kernel(b, d)(target_bbox.reshape(-1), tgt2d, pe2d)
    ctx_out = _ctx_add(context_tokens, positional_encoding, bn=16)
    return ctx_out, tgt_out.reshape(1, b, d)


# speedup vs baseline: 1.0065x; 1.0065x over previous
import functools

import jax
import jax.numpy as jnp
from jax import lax
from jax.experimental import pallas as pl
from jax.experimental.pallas import tpu as pltpu
from jax.experimental.pallas import tpu_sc as plsc

_TPD = 24.0


def _ctx_add_body(ctx_ref, pe_ref, out_ref):
    out_ref[...] = ctx_ref[...] + pe_ref[...]


def _ctx_add(context_tokens, positional_encoding, bn):
    n, b, d = context_tokens.shape
    return pl.pallas_call(
        _ctx_add_body,
        grid=(n // bn,),
        in_specs=[
            pl.BlockSpec((bn, b, d), lambda i: (i, 0, 0)),
            pl.BlockSpec((bn, 1, d), lambda i: (i, 0, 0)),
        ],
        out_specs=pl.BlockSpec((bn, b, d), lambda i: (i, 0, 0)),
        out_shape=jax.ShapeDtypeStruct((n, b, d), jnp.float32),
        compiler_params=pltpu.CompilerParams(
            vmem_limit_bytes=128 * 1024 * 1024),
    )(context_tokens, positional_encoding)


def _make_tgt_kernel(B, D):
    L = 16
    n_workers = B // L
    mesh = plsc.VectorSubcoreMesh(
        core_axis_name="c", subcore_axis_name="s", num_cores=1)

    @functools.partial(
        pl.kernel,
        mesh=mesh,
        out_type=jax.ShapeDtypeStruct((B, D), jnp.float32),
        scratch_types=[
            pltpu.VMEM((4, L), jnp.float32),
            pltpu.VMEM((L,), jnp.int32),
            pltpu.VMEM((L, D), jnp.float32),
            pltpu.VMEM((L, D), jnp.float32),
            pltpu.SemaphoreType.DMA,
            pltpu.SemaphoreType.DMA,
        ],
    )
    def tgt_kernel(bbox_hbm, tgt_hbm, pe_hbm, out_hbm,
                   bbox_v, idx_v, rows_v, tgt_v, sem_g, sem_t):
        wid = lax.axis_index("s")

        @pl.when(wid < n_workers)
        def _():
            base = wid * L
            tcp = pltpu.async_copy(tgt_hbm.at[pl.ds(base, L)], tgt_v, sem_t)
            bcp = [
                pltpu.async_copy(
                    bbox_hbm.at[j, pl.ds(base, L)], bbox_v.at[j], sem_g)
                for j in range(4)
            ]
            for c in bcp:
                c.wait()
            x = bbox_v[0, :]
            y = bbox_v[1, :]
            w = bbox_v[2, :]
            h = bbox_v[3, :]
            cx = (x + w / 2.0) * _TPD
            cy = (y + h / 2.0) * _TPD
            tx = cx.astype(jnp.int32)
            ty = cy.astype(jnp.int32)
            ix = jnp.where(cx > tx.astype(jnp.float32), tx, tx - 1)
            iy = jnp.where(cy > ty.astype(jnp.float32), ty, ty - 1)
            idx_v[...] = ix + iy * 24

            pltpu.async_copy(pe_hbm.at[idx_v], rows_v, sem_g).wait()
            tcp.wait()

            n_chunks = L * D // L
            UNROLL = 1

            def add_body(i, _):
                for u in range(UNROLL):
                    k = i * UNROLL + u
                    r = k // (D // L)
                    off = (k % (D // L)) * L
                    rows_v[r, pl.ds(off, L)] = (
                        rows_v[r, pl.ds(off, L)] + tgt_v[r, pl.ds(off, L)]
                    )
                return 0
            lax.fori_loop(0, n_chunks // UNROLL, add_body, 0)

            pltpu.sync_copy(rows_v, out_hbm.at[pl.ds(base, L)])

    return tgt_kernel


def kernel(context_tokens, target_tokens, target_bbox, positional_encoding):
    n, b, d = context_tokens.shape
    pe2d = positional_encoding.reshape(n, d)
    tgt2d = target_tokens.reshape(b, d)
    tgt_out = _make_tgt_kernel(b, d)(target_bbox.T, tgt2d, pe2d)
    ctx_out = _ctx_add(context_tokens, positional_encoding, bn=18)
    return ctx_out, tgt_out.reshape(1, b, d)
